# Initial kernel scaffold; baseline (speedup 1.0000x reference)
import jax
import jax.numpy as jnp
from jax.experimental import pallas as pl


def kernel(A, B):
    raise NotImplementedError("write your pallas kernel here")


# speedup vs baseline: 1.2231x; 1.2231x over previous
import functools

import jax
import jax.numpy as jnp
from jax.experimental import pallas as pl
from jax.experimental.pallas import tpu as pltpu

_BN = 256


def _gemm_body(a_ref, bw_ref, o_ref):
    bw = bw_ref[...]
    kw = bw.shape[1]
    slabs = []
    for c in range(16):
        i, j = c // 4, c % 4
        s = 8 * j + 2 * i
        t = (bw >> jnp.uint32(s)) & jnp.uint32(3)
        slabs.append(t.astype(jnp.bfloat16))
    w = jnp.concatenate(slabs, axis=1)
    acc = jax.lax.dot_general(
        a_ref[...], w,
        dimension_numbers=(((1,), (1,)), ((), ())),
        preferred_element_type=jnp.float32,
    )
    o_ref[...] = acc.astype(jnp.int32)


@jax.jit
def kernel(A, B):
    M, K = A.shape
    N = B.shape[0]
    A_perm = (
        A.reshape(M, K // 16, 16).transpose(0, 2, 1).reshape(M, K)
        .astype(jnp.bfloat16)
    )
    Bw = jax.lax.bitcast_convert_type(B.reshape(N, K // 16, 4), jnp.uint32)

    grid = (N // _BN,)
    return pl.pallas_call(
        _gemm_body,
        grid=grid,
        in_specs=[
            pl.BlockSpec((M, K), lambda n: (0, 0)),
            pl.BlockSpec((_BN, K // 16), lambda n: (n, 0)),
        ],
        out_specs=pl.BlockSpec((M, _BN), lambda n: (0, n)),
        out_shape=jax.ShapeDtypeStruct((M, N), jnp.int32),
        compiler_params=pltpu.CompilerParams(
            dimension_semantics=("parallel",),
        ),
    )(A_perm, Bw)


# speedup vs baseline: 2.4609x; 2.0120x over previous
import jax
import jax.numpy as jnp
from jax.experimental import pallas as pl
from jax.experimental.pallas import tpu as pltpu

_BN = 256


def _gemm_body(a_ref, b_ref, o_ref):
    b = b_ref[...]
    b32 = pltpu.bitcast(b, jnp.int32)
    mask = jnp.int32(0x03030303)
    slabs = []
    for i in range(4):
        t32 = (b32 >> (2 * i)) & mask
        t = pltpu.bitcast(t32, jnp.int8)
        slabs.append(t.astype(jnp.bfloat16))
    w = jnp.concatenate(slabs, axis=1)
    acc = jax.lax.dot_general(
        a_ref[...], w,
        dimension_numbers=(((1,), (1,)), ((), ())),
        preferred_element_type=jnp.float32,
    )
    o_ref[...] = acc.astype(jnp.int32)


@jax.jit
def kernel(A, B):
    M, K = A.shape
    N = B.shape[0]
    A_perm = (
        A.reshape(M, K // 16, 4, 4).transpose(0, 2, 1, 3).reshape(M, K)
        .astype(jnp.bfloat16)
    )

    grid = (N // _BN,)
    return pl.pallas_call(
        _gemm_body,
        grid=grid,
        in_specs=[
            pl.BlockSpec((M, K), lambda n: (0, 0)),
            pl.BlockSpec((_BN, K // 4), lambda n: (n, 0)),
        ],
        out_specs=pl.BlockSpec((M, _BN), lambda n: (0, n)),
        out_shape=jax.ShapeDtypeStruct((M, N), jnp.int32),
        compiler_params=pltpu.CompilerParams(
            dimension_semantics=("parallel",),
        ),
    )(A_perm, B)


# speedup vs baseline: 3.0211x; 1.2277x over previous
import jax
import jax.numpy as jnp
from jax.experimental import pallas as pl
from jax.experimental.pallas import tpu as pltpu

_BN = 256


def _gemm_body(a0_ref, a1_ref, a2_ref, a3_ref, b_ref, o_ref, a_bf16_ref):
    kq = a0_ref.shape[1]

    @pl.when(pl.program_id(0) == 0)
    def _():
        for i, ref in enumerate((a0_ref, a1_ref, a2_ref, a3_ref)):
            a_bf16_ref[:, i * kq:(i + 1) * kq] = ref[...].astype(jnp.bfloat16)

    b = b_ref[...]
    b32 = pltpu.bitcast(b, jnp.int32)
    mask = jnp.int32(0x03030303)
    slabs = []
    for i in range(4):
        t32 = (b32 >> (2 * i)) & mask
        t = pltpu.bitcast(t32, jnp.int8)
        slabs.append(t.astype(jnp.bfloat16))
    w = jnp.concatenate(slabs, axis=1)

    acc = jax.lax.dot_general(
        a_bf16_ref[...], w,
        dimension_numbers=(((1,), (1,)), ((), ())),
        preferred_element_type=jnp.float32,
    )
    o_ref[...] = acc.astype(jnp.int32)


@jax.jit
def kernel(A, B):
    M, K = A.shape
    N = B.shape[0]
    A4 = A.reshape(M, K // 16, 4, 4)
    a_slabs = [A4[:, :, i, :].reshape(M, K // 4) for i in range(4)]

    grid = (N // _BN,)
    a_spec = pl.BlockSpec((M, K // 4), lambda n: (0, 0))
    return pl.pallas_call(
        _gemm_body,
        grid=grid,
        in_specs=[
            a_spec, a_spec, a_spec, a_spec,
            pl.BlockSpec((_BN, K // 4), lambda n: (n, 0)),
        ],
        out_specs=pl.BlockSpec((M, _BN), lambda n: (0, n)),
        out_shape=jax.ShapeDtypeStruct((M, N), jnp.int32),
        scratch_shapes=[pltpu.VMEM((M, K), jnp.bfloat16)],
        compiler_params=pltpu.CompilerParams(
            dimension_semantics=("arbitrary",),
        ),
    )(*a_slabs, B)


# speedup vs baseline: 3.0951x; 1.0245x over previous
import jax
import jax.numpy as jnp
from jax.experimental import pallas as pl
from jax.experimental.pallas import tpu as pltpu

_BN = 512


def _gemm_body(a0_ref, a1_ref, a2_ref, a3_ref, b_ref, o_ref, a_bf16_ref):
    kq = a0_ref.shape[1]

    @pl.when(pl.program_id(0) == 0)
    def _():
        for i, ref in enumerate((a0_ref, a1_ref, a2_ref, a3_ref)):
            a_bf16_ref[:, i * kq:(i + 1) * kq] = ref[...].astype(jnp.bfloat16)

    b = b_ref[...]
    b32 = pltpu.bitcast(b, jnp.int32)
    mask = jnp.int32(0x03030303)
    slabs = []
    for i in range(4):
        t32 = (b32 >> (2 * i)) & mask
        t = pltpu.bitcast(t32, jnp.int8)
        slabs.append(t.astype(jnp.bfloat16))
    w = jnp.concatenate(slabs, axis=1)

    acc = jax.lax.dot_general(
        a_bf16_ref[...], w,
        dimension_numbers=(((1,), (1,)), ((), ())),
        preferred_element_type=jnp.float32,
    )
    o_ref[...] = acc.astype(jnp.int32)


@jax.jit
def kernel(A, B):
    M, K = A.shape
    N = B.shape[0]
    A4 = A.reshape(M, K // 16, 4, 4)
    a_slabs = [A4[:, :, i, :].reshape(M, K // 4) for i in range(4)]

    grid = (pl.cdiv(N, _BN),)
    a_spec = pl.BlockSpec((M, K // 4), lambda n: (0, 0))
    return pl.pallas_call(
        _gemm_body,
        grid=grid,
        in_specs=[
            a_spec, a_spec, a_spec, a_spec,
            pl.BlockSpec((_BN, K // 4), lambda n: (n, 0)),
        ],
        out_specs=pl.BlockSpec((M, _BN), lambda n: (0, n)),
        out_shape=jax.ShapeDtypeStruct((M, N), jnp.int32),
        scratch_shapes=[pltpu.VMEM((M, K), jnp.bfloat16)],
        compiler_params=pltpu.CompilerParams(
            dimension_semantics=("arbitrary",),
        ),
    )(*a_slabs, B)


# speedup vs baseline: 3.1471x; 1.0168x over previous
import jax
import jax.numpy as jnp
from jax.experimental import pallas as pl
from jax.experimental.pallas import tpu as pltpu

_BN = 1024


def _gemm_body(a0_ref, a1_ref, a2_ref, a3_ref, b_ref, o_ref, a_bf16_ref):
    kq = a0_ref.shape[1]

    @pl.when(pl.program_id(0) == 0)
    def _():
        for i, ref in enumerate((a0_ref, a1_ref, a2_ref, a3_ref)):
            a_bf16_ref[:, i * kq:(i + 1) * kq] = ref[...].astype(jnp.bfloat16)

    b = b_ref[...]
    b32 = pltpu.bitcast(b, jnp.int32)
    mask = jnp.int32(0x03030303)
    slabs = []
    for i in range(4):
        t32 = (b32 >> (2 * i)) & mask
        t = pltpu.bitcast(t32, jnp.int8)
        slabs.append(t.astype(jnp.bfloat16))
    w = jnp.concatenate(slabs, axis=1)

    acc = jax.lax.dot_general(
        a_bf16_ref[...], w,
        dimension_numbers=(((1,), (1,)), ((), ())),
        preferred_element_type=jnp.float32,
    )
    o_ref[...] = acc.astype(jnp.int32)


@jax.jit
def kernel(A, B):
    M, K = A.shape
    N = B.shape[0]
    A4 = A.reshape(M, K // 16, 4, 4)
    a_slabs = [A4[:, :, i, :].reshape(M, K // 4) for i in range(4)]

    grid = (pl.cdiv(N, _BN),)
    a_spec = pl.BlockSpec((M, K // 4), lambda n: (0, 0))
    return pl.pallas_call(
        _gemm_body,
        grid=grid,
        in_specs=[
            a_spec, a_spec, a_spec, a_spec,
            pl.BlockSpec((_BN, K // 4), lambda n: (n, 0)),
        ],
        out_specs=pl.BlockSpec((M, _BN), lambda n: (0, n)),
        out_shape=jax.ShapeDtypeStruct((M, N), jnp.int32),
        scratch_shapes=[pltpu.VMEM((M, K), jnp.bfloat16)],
        compiler_params=pltpu.CompilerParams(
            dimension_semantics=("arbitrary",),
        ),
    )(*a_slabs, B)


# speedup vs baseline: 4.1354x; 1.3140x over previous
import jax
import jax.numpy as jnp
from jax.experimental import pallas as pl
from jax.experimental.pallas import tpu as pltpu

_BN = 1024


def _permute_a(a8):
    a32 = pltpu.bitcast(a8, jnp.int32)
    rows, kk = a32.shape
    nch = kk // 128
    l2 = jax.lax.broadcasted_iota(jnp.int32, (rows, kk), 1)
    cpos = (l2 // 128) % 4
    l3 = jax.lax.broadcasted_iota(jnp.int32, (rows, nch, 128), 2)
    c3 = jax.lax.broadcasted_iota(jnp.int32, (rows, nch, 128), 1)
    lane_idx = 16 * ((l3 // 4) % 8) + 4 * (c3 % 4) + (l3 % 4)
    parts = []
    for r in range(4):
        rolls = [pltpu.roll(a32, ((i - r) % (kk // 128)) * 128, axis=1) for i in range(4)]
        v = jnp.where(
            cpos == 0, rolls[0],
            jnp.where(cpos == 1, rolls[1],
                      jnp.where(cpos == 2, rolls[2], rolls[3])))
        v3 = v.reshape(rows, nch, 128)
        parts.append(jnp.take_along_axis(v3, lane_idx, axis=2))
    band = l3 // 32
    merged = jnp.where(
        band == 0, parts[0],
        jnp.where(band == 1, parts[1],
                  jnp.where(band == 2, parts[2], parts[3])))
    return pltpu.bitcast(merged.reshape(rows, kk), jnp.int8)


def _gemm_body(a_ref, b_ref, o_ref, a_bf16_ref):
    @pl.when(pl.program_id(0) == 0)
    def _():
        a_bf16_ref[...] = _permute_a(a_ref[...]).astype(jnp.bfloat16)

    b = b_ref[...]
    b32 = pltpu.bitcast(b, jnp.int32)
    mask = jnp.int32(0x03030303)
    slabs = []
    for i in range(4):
        t32 = (b32 >> (2 * i)) & mask
        t = pltpu.bitcast(t32, jnp.int8)
        slabs.append(t.astype(jnp.bfloat16))
    kq = b.shape[1]
    pieces = []
    for q in range(kq // 128):
        for i in range(4):
            pieces.append(slabs[i][:, q * 128:(q + 1) * 128])
    w = jnp.concatenate(pieces, axis=1)

    acc = jax.lax.dot_general(
        a_bf16_ref[...], w,
        dimension_numbers=(((1,), (1,)), ((), ())),
        preferred_element_type=jnp.float32,
    )
    o_ref[...] = acc.astype(jnp.int32)


@jax.jit
def kernel(A, B):
    M, K = A.shape
    N = B.shape[0]
    grid = (pl.cdiv(N, _BN),)
    return pl.pallas_call(
        _gemm_body,
        grid=grid,
        in_specs=[
            pl.BlockSpec((M, K), lambda n: (0, 0)),
            pl.BlockSpec((_BN, K // 4), lambda n: (n, 0)),
        ],
        out_specs=pl.BlockSpec((M, _BN), lambda n: (0, n)),
        out_shape=jax.ShapeDtypeStruct((M, N), jnp.int32),
        scratch_shapes=[pltpu.VMEM((M, K), jnp.bfloat16)],
        compiler_params=pltpu.CompilerParams(
            dimension_semantics=("arbitrary",),
        ),
    )(A, B)
